# Initial kernel scaffold; baseline (speedup 1.0000x reference)
import jax
import jax.numpy as jnp
from jax.experimental import pallas as pl


def kernel(xyz, points, W1, W2, W3, gamma1, gamma2, gamma3, beta1, beta2, beta3):
    raise NotImplementedError("write your pallas kernel here")


# speedup vs baseline: 13.3270x; 13.3270x over previous
import functools

import jax
import jax.numpy as jnp
import numpy as np
from jax import lax
from jax.experimental import pallas as pl
from jax.experimental.pallas import tpu as pltpu
from jax.experimental.pallas import tpu_sc as plsc

_B = 8
_N = 4096
_NPOINT = 512
_NSAMPLE = 32
_C1 = 128
_C2 = 128
_C3 = 256
_INV_STD = float(1.0 / np.sqrt(1.0 + 1e-3))
_ROWS = _B * _NPOINT * _NSAMPLE
_TILE_C = 64
_TILE_R = _TILE_C * _NSAMPLE


def _fps_body(xyzt_ref, lanes_ref, lanesp_ref, idx_ref, nx_ref):
    x0 = xyzt_ref[0]
    x1 = xyzt_ref[1]
    x2 = xyzt_ref[2]
    lanes = lanes_ref[...]
    lanes_p = lanesp_ref[...]

    def body(i, carry):
        dist, far, idxa, c0a, c1a, c2a = carry
        oh = (lanes == far).astype(jnp.float32)
        c0 = jnp.sum(x0 * oh, axis=1, keepdims=True)
        c1 = jnp.sum(x1 * oh, axis=1, keepdims=True)
        c2 = jnp.sum(x2 * oh, axis=1, keepdims=True)
        seli = (lanes_p == i).astype(jnp.int32)
        self = seli.astype(jnp.float32)
        idxa = idxa + seli * far
        c0a = c0a + self * c0
        c1a = c1a + self * c1
        c2a = c2a + self * c2
        d = jnp.square(x0 - c0) + jnp.square(x1 - c1) + jnp.square(x2 - c2)
        dist = jnp.minimum(dist, d)
        mx = jnp.max(dist, axis=1, keepdims=True)
        far = jnp.min(jnp.where(dist == mx, lanes, jnp.int32(_N)),
                      axis=1, keepdims=True)
        return dist, far, idxa, c0a, c1a, c2a

    dist0 = jnp.full((_B, _N), 1e10, jnp.float32)
    far0 = jnp.zeros((_B, 1), jnp.int32)
    zp = jnp.zeros((_B, _NPOINT), jnp.float32)
    _, _, idxa, c0a, c1a, c2a = lax.fori_loop(
        0, _NPOINT, body,
        (dist0, far0, jnp.zeros((_B, _NPOINT), jnp.int32), zp, zp, zp))
    idx_ref[...] = idxa
    nx_ref[0] = c0a
    nx_ref[1] = c1a
    nx_ref[2] = c2a


def _prep_body(xyz_ref, xyzt_ref, nx_ref, pts_ref, w1a_ref, w1b_ref,
               g1_ref, b1_ref, lanes_ref, laness_ref,
               t_ref, bias1_ref, idx_ref, s_ref):
    b = pl.program_id(0)
    xyz = xyz_ref[0]
    xyzt = xyzt_ref[0]
    nx = nx_ref[0]
    pts = pts_ref[0]
    w1a = w1a_ref[...]
    w1b = w1b_ref[...]

    t_ref[0] = (jnp.dot(xyz, w1a, preferred_element_type=jnp.float32)
                + jnp.dot(pts, w1b, preferred_element_type=jnp.float32))

    c1 = jnp.dot(nx, w1a, preferred_element_type=jnp.float32)
    a1 = g1_ref[...] * _INV_STD
    bias1_ref[0] = b1_ref[...] - a1 * c1

    dot3 = jnp.dot(nx, xyzt, preferred_element_type=jnp.float32)
    csq = jnp.sum(nx * nx, axis=1, keepdims=True)
    xsq = jnp.sum(xyzt * xyzt, axis=0, keepdims=True)
    s_ref[...] = (csq - 2.0 * dot3) + xsq

    lanes = lanes_ref[...]
    lanes_s = laness_ref[...]
    base = b * _N

    def body(k, idxa):
        s = s_ref[...]
        mn = jnp.min(s, axis=1, keepdims=True)
        am = jnp.min(jnp.where(s == mn, lanes, jnp.int32(_N)),
                     axis=1, keepdims=True)
        s_ref[...] = jnp.where(lanes == am, jnp.float32(float("inf")), s)
        return idxa + (lanes_s == k).astype(jnp.int32) * (am + base)

    idx_ref[0] = lax.fori_loop(
        0, _NSAMPLE, body, jnp.zeros((_NPOINT, _NSAMPLE), jnp.int32))


def _sc_gather(table, idx3):
    info = plsc.get_sparse_core_info()
    nc = info.num_cores
    rows_per_w = _ROWS // (nc * info.num_subcores)
    n_chunks = rows_per_w // 128
    mesh = plsc.VectorSubcoreMesh(core_axis_name="c", subcore_axis_name="s")

    @functools.partial(
        pl.kernel, mesh=mesh,
        out_type=jax.ShapeDtypeStruct((_ROWS, _C1), jnp.float32),
        scratch_types=[
            pltpu.VMEM((n_chunks, 128), jnp.int32),
            pltpu.VMEM((128, _C1), jnp.float32),
            pltpu.SemaphoreType.DMA,
        ],
    )
    def k(table_hbm, idx_hbm, out_hbm, idx_v, rows_v, sem):
        wid = lax.axis_index("s") * nc + lax.axis_index("c")
        pltpu.sync_copy(idx_hbm.at[wid], idx_v)
        base = wid * rows_per_w

        def chunk(c, _):
            pltpu.async_copy(table_hbm.at[idx_v.at[c]], rows_v, sem).wait()
            pltpu.sync_copy(rows_v, out_hbm.at[pl.ds(base + c * 128, 128)])
            return 0

        lax.fori_loop(0, n_chunks, chunk, 0)

    return k(table, idx3)


def _mlp_tile(g_ref, bias1_ref, w2_ref, w3_ref, g1_ref, g2_ref, b2_ref,
              g3_ref, b3_ref, out_ref):
    rows = g_ref[...]
    a1 = g1_ref[...] * _INV_STD
    bias1 = bias1_ref[...]
    b1r = jnp.reshape(
        jnp.broadcast_to(bias1[:, None, :], (_TILE_C, _NSAMPLE, _C1)),
        (_TILE_R, _C1))
    h = jnp.maximum(rows * a1 + b1r, 0.0)
    h = jnp.dot(h, w2_ref[...], preferred_element_type=jnp.float32)
    h = jnp.maximum(h * (g2_ref[...] * _INV_STD) + b2_ref[...], 0.0)
    h = jnp.dot(h, w3_ref[...], preferred_element_type=jnp.float32)
    h = jnp.maximum(h * (g3_ref[...] * _INV_STD) + b3_ref[...], 0.0)
    out_ref[...] = jnp.max(
        jnp.reshape(h, (_TILE_C, _NSAMPLE, _C3)), axis=1)


def kernel(xyz, points, W1, W2, W3, gamma1, gamma2, gamma3,
           beta1, beta2, beta3):
    f32 = jnp.float32
    xyzt3 = jnp.transpose(xyz, (2, 0, 1))
    xyzt = jnp.swapaxes(xyz, 1, 2)

    lanes_n = jnp.broadcast_to(jnp.arange(_N, dtype=jnp.int32), (_B, _N))
    lanes_p = jnp.broadcast_to(jnp.arange(_NPOINT, dtype=jnp.int32),
                               (_B, _NPOINT))
    fps_idx, nxT = pl.pallas_call(
        _fps_body,
        out_shape=(jax.ShapeDtypeStruct((_B, _NPOINT), jnp.int32),
                   jax.ShapeDtypeStruct((3, _B, _NPOINT), f32)),
    )(xyzt3, lanes_n, lanes_p)
    del fps_idx
    new_xyz = jnp.transpose(nxT, (1, 2, 0))

    w1a = W1[:3]
    w1b = W1[3:]
    g1r = gamma1.reshape(1, _C1)
    b1r = beta1.reshape(1, _C1)

    T, bias1, idx = pl.pallas_call(
        _prep_body,
        grid=(_B,),
        in_specs=[
            pl.BlockSpec((1, _N, 3), lambda b: (b, 0, 0)),
            pl.BlockSpec((1, 3, _N), lambda b: (b, 0, 0)),
            pl.BlockSpec((1, _NPOINT, 3), lambda b: (b, 0, 0)),
            pl.BlockSpec((1, _N, 64), lambda b: (b, 0, 0)),
            pl.BlockSpec((3, _C1), lambda b: (0, 0)),
            pl.BlockSpec((64, _C1), lambda b: (0, 0)),
            pl.BlockSpec((1, _C1), lambda b: (0, 0)),
            pl.BlockSpec((1, _C1), lambda b: (0, 0)),
            pl.BlockSpec((_NPOINT, _N), lambda b: (0, 0)),
            pl.BlockSpec((_NPOINT, _NSAMPLE), lambda b: (0, 0)),
        ],
        out_specs=[
            pl.BlockSpec((1, _N, _C1), lambda b: (b, 0, 0)),
            pl.BlockSpec((1, _NPOINT, _C1), lambda b: (b, 0, 0)),
            pl.BlockSpec((1, _NPOINT, _NSAMPLE), lambda b: (b, 0, 0)),
        ],
        out_shape=(jax.ShapeDtypeStruct((_B, _N, _C1), f32),
                   jax.ShapeDtypeStruct((_B, _NPOINT, _C1), f32),
                   jax.ShapeDtypeStruct((_B, _NPOINT, _NSAMPLE), jnp.int32)),
        scratch_shapes=[pltpu.VMEM((_NPOINT, _N), f32)],
    )(xyz, xyzt, new_xyz, points, w1a, w1b, g1r, b1r,
      jnp.broadcast_to(jnp.arange(_N, dtype=jnp.int32), (_NPOINT, _N)),
      jnp.broadcast_to(jnp.arange(_NSAMPLE, dtype=jnp.int32),
                       (_NPOINT, _NSAMPLE)))

    table = T.reshape(_B * _N, _C1)
    idx3 = idx.reshape(32, _ROWS // (32 * 128), 128)
    gathered = _sc_gather(table, idx3)

    bias1f = bias1.reshape(_B * _NPOINT, _C1)
    n_tiles = _ROWS // _TILE_R

    pooled = pl.pallas_call(
        _mlp_tile,
        grid=(n_tiles,),
        in_specs=[
            pl.BlockSpec((_TILE_R, _C1), lambda t: (t, 0)),
            pl.BlockSpec((_TILE_C, _C1), lambda t: (t, 0)),
            pl.BlockSpec((_C1, _C2), lambda t: (0, 0)),
            pl.BlockSpec((_C2, _C3), lambda t: (0, 0)),
            pl.BlockSpec((1, _C1), lambda t: (0, 0)),
            pl.BlockSpec((1, _C2), lambda t: (0, 0)),
            pl.BlockSpec((1, _C2), lambda t: (0, 0)),
            pl.BlockSpec((1, _C3), lambda t: (0, 0)),
            pl.BlockSpec((1, _C3), lambda t: (0, 0)),
        ],
        out_specs=pl.BlockSpec((_TILE_C, _C3), lambda t: (t, 0)),
        out_shape=jax.ShapeDtypeStruct((_B * _NPOINT, _C3), f32),
    )(gathered, bias1f, W2, W3, g1r, gamma2.reshape(1, _C2),
      beta2.reshape(1, _C2), gamma3.reshape(1, _C3), beta3.reshape(1, _C3))

    new_points = pooled.reshape(_B, _NPOINT, _C3)
    return (new_xyz, new_points)
